# baseline (speedup 1.0000x reference)
import functools

import jax
import jax.numpy as jnp
from jax import lax
from jax.experimental import pallas as pl
from jax.experimental.pallas import tpu as pltpu
from jax.experimental.pallas import tpu_sc as plsc

_R_BLK = 256


def _sim_argmax_body(c_ref, wa_ref, wb_ref, idx_ref, val_ref):
    k = pl.program_id(0)
    B = c_ref.shape[0]
    G = wa_ref.shape[1]
    D = c_ref.shape[1]

    @pl.when(k == 0)
    def _init():
        val_ref[...] = jnp.full((B,), -jnp.inf, dtype=jnp.float32)
        idx_ref[...] = jnp.zeros((B,), dtype=jnp.int32)

    c = c_ref[...]
    rn = jnp.sqrt(jnp.sum(c * c, axis=1))
    cs = c / jnp.maximum(rn, 1e-12)[:, None]
    wt = jnp.concatenate(
        [jnp.transpose(wa_ref[...], (1, 0, 2)),
         jnp.transpose(wb_ref[...], (1, 0, 2))], axis=1)
    for q in range(G):
        w = wt[q]
        cn = jnp.sqrt(jnp.sum(w * w, axis=0))
        wn = w / jnp.maximum(cn, 1e-12)[None, :]
        sim = jnp.dot(cs, wn, preferred_element_type=jnp.float32)
        lmax = jnp.max(sim, axis=1)
        lidx = jnp.argmax(sim, axis=1).astype(jnp.int32) + (q * D + k * _R_BLK)
        better = lmax > val_ref[...]
        idx_ref[...] = jnp.where(better, lidx, idx_ref[...])
        val_ref[...] = jnp.where(better, lmax, val_ref[...])


def _sim_argmax(content, bc3):
    B, D = content.shape
    G = bc3.shape[1]
    return pl.pallas_call(
        _sim_argmax_body,
        grid=(D // _R_BLK,),
        in_specs=[
            pl.BlockSpec((B, D), lambda k: (0, 0)),
            pl.BlockSpec((D // 2, G, _R_BLK), lambda k: (0, 0, k)),
            pl.BlockSpec((D // 2, G, _R_BLK), lambda k: (1, 0, k)),
        ],
        out_specs=pl.BlockSpec((B,), lambda k: (0,)),
        out_shape=jax.ShapeDtypeStruct((B,), jnp.int32),
        scratch_shapes=[pltpu.VMEM((B,), jnp.float32)],
    )(content, bc3, bc3)


def _gather_rows_sc(table, idx):
    B = idx.shape[0]
    D = table.shape[1]
    n_workers = 16
    b_per_w = B // n_workers
    mesh = plsc.VectorSubcoreMesh(core_axis_name="c", subcore_axis_name="s")

    @functools.partial(
        pl.kernel,
        mesh=mesh,
        out_type=jax.ShapeDtypeStruct((B, D), jnp.float32),
        scratch_types=[
            pltpu.VMEM((b_per_w,), jnp.int32),
            pltpu.VMEM((b_per_w, D), jnp.float32),
            pltpu.SemaphoreType.DMA,
        ],
    )
    def k(table_hbm, idx_hbm, out_hbm, idx_v, rows_v, sem):
        wid = lax.axis_index("s") * 2 + lax.axis_index("c")

        @pl.when(wid < n_workers)
        def _():
            base = wid * b_per_w
            pltpu.sync_copy(idx_hbm.at[pl.ds(base, b_per_w)], idx_v)
            pltpu.async_copy(table_hbm.at[idx_v], rows_v, sem).wait()
            pltpu.sync_copy(rows_v, out_hbm.at[pl.ds(base, b_per_w)])

    return k(table, idx)


def kernel(content, bank_content, bank_style):
    B, D = content.shape
    M = bank_content.shape[0]
    G = M // D
    bc3 = bank_content.reshape(D, G, D)
    idx = _sim_argmax(content, bc3)
    return _gather_rows_sc(bank_style, idx)


# speedup vs baseline: 1.0974x; 1.0974x over previous
import functools

import jax
import jax.numpy as jnp
from jax import lax
from jax.experimental import pallas as pl
from jax.experimental.pallas import tpu as pltpu
from jax.experimental.pallas import tpu_sc as plsc

_R_BLK = 128


def _sim_argmax_body(c_ref, w_ref, idx_ref, val_ref):
    k = pl.program_id(0)
    B = c_ref.shape[0]
    G = w_ref.shape[1]
    D = c_ref.shape[1]

    @pl.when(k == 0)
    def _init():
        val_ref[...] = jnp.full((B,), -jnp.inf, dtype=jnp.float32)
        idx_ref[...] = jnp.zeros((B,), dtype=jnp.int32)

    c = c_ref[...]
    rn = jnp.sqrt(jnp.sum(c * c, axis=1))
    cs = c / jnp.maximum(rn, 1e-12)[:, None]
    w3 = w_ref[...]
    cn = jnp.maximum(jnp.sqrt(jnp.sum(w3 * w3, axis=0)), 1e-12)
    wn3 = (w3 / cn[None, :, :]).astype(jnp.bfloat16)
    wt = jnp.transpose(wn3, (1, 0, 2))
    for q in range(G):
        sim = lax.dot_general(cs, wt[q], (((1,), (0,)), ((), ())),
                              preferred_element_type=jnp.float32)
        lmax = jnp.max(sim, axis=1)
        lidx = jnp.argmax(sim, axis=1).astype(jnp.int32) + (q * D + k * _R_BLK)
        better = lmax > val_ref[...]
        idx_ref[...] = jnp.where(better, lidx, idx_ref[...])
        val_ref[...] = jnp.where(better, lmax, val_ref[...])


def _sim_argmax(content, bc3):
    B, D = content.shape
    G = bc3.shape[1]
    return pl.pallas_call(
        _sim_argmax_body,
        grid=(D // _R_BLK,),
        in_specs=[
            pl.BlockSpec((B, D), lambda k: (0, 0)),
            pl.BlockSpec((D, G, _R_BLK), lambda k: (0, 0, k)),
        ],
        out_specs=pl.BlockSpec((B,), lambda k: (0,)),
        out_shape=jax.ShapeDtypeStruct((B,), jnp.int32),
        scratch_shapes=[pltpu.VMEM((B,), jnp.float32)],
    )(content, bc3)


def _gather_rows_sc(table, idx):
    B = idx.shape[0]
    D = table.shape[1]
    n_workers = 16
    b_per_w = B // n_workers
    mesh = plsc.VectorSubcoreMesh(core_axis_name="c", subcore_axis_name="s")

    @functools.partial(
        pl.kernel,
        mesh=mesh,
        out_type=jax.ShapeDtypeStruct((B, D), jnp.float32),
        scratch_types=[
            pltpu.VMEM((b_per_w,), jnp.int32),
            pltpu.VMEM((b_per_w, D), jnp.float32),
            pltpu.SemaphoreType.DMA,
        ],
    )
    def k(table_hbm, idx_hbm, out_hbm, idx_v, rows_v, sem):
        wid = lax.axis_index("s") * 2 + lax.axis_index("c")

        @pl.when(wid < n_workers)
        def _():
            base = wid * b_per_w
            pltpu.sync_copy(idx_hbm.at[pl.ds(base, b_per_w)], idx_v)
            pltpu.async_copy(table_hbm.at[idx_v], rows_v, sem).wait()
            pltpu.sync_copy(rows_v, out_hbm.at[pl.ds(base, b_per_w)])

    return k(table, idx)


def kernel(content, bank_content, bank_style):
    B, D = content.shape
    M = bank_content.shape[0]
    G = M // D
    bc3 = bank_content.reshape(D, G, D)
    idx = _sim_argmax(content, bc3)
    return _gather_rows_sc(bank_style, idx)


# speedup vs baseline: 1.7469x; 1.5919x over previous
import functools

import jax
import jax.numpy as jnp
from jax import lax
from jax.experimental import pallas as pl
from jax.experimental.pallas import tpu as pltpu
from jax.experimental.pallas import tpu_sc as plsc

_R_BLK = 128


def _sim_argmax_body(c_ref, w_ref, idx_ref, val_ref):
    k = pl.program_id(0)
    B = c_ref.shape[0]
    G = w_ref.shape[1]
    D = c_ref.shape[1]

    @pl.when(k == 0)
    def _init():
        val_ref[...] = jnp.full((B,), -jnp.inf, dtype=jnp.float32)
        idx_ref[...] = jnp.zeros((B,), dtype=jnp.int32)

    c = c_ref[...]
    rn = jnp.sqrt(jnp.sum(c * c, axis=1))
    cs = c / jnp.maximum(rn, 1e-12)[:, None]
    w2 = w_ref[...].reshape(D, G * _R_BLK)
    cn = jnp.sqrt(jnp.sum(w2 * w2, axis=0))
    wn = w2 / jnp.maximum(cn, 1e-12)[None, :]
    sim = jnp.dot(cs, wn, preferred_element_type=jnp.float32)
    lmax = jnp.max(sim, axis=1)
    j = jnp.argmax(sim, axis=1).astype(jnp.int32)
    lidx = (j // _R_BLK) * D + (j % _R_BLK) + k * _R_BLK
    better = lmax > val_ref[...]
    idx_ref[...] = jnp.where(better, lidx, idx_ref[...])
    val_ref[...] = jnp.where(better, lmax, val_ref[...])


def _sim_argmax(content, bc3):
    B, D = content.shape
    G = bc3.shape[1]
    return pl.pallas_call(
        _sim_argmax_body,
        grid=(D // _R_BLK,),
        in_specs=[
            pl.BlockSpec((B, D), lambda k: (0, 0)),
            pl.BlockSpec((D, G, _R_BLK), lambda k: (0, 0, k)),
        ],
        out_specs=pl.BlockSpec((B,), lambda k: (0,)),
        out_shape=jax.ShapeDtypeStruct((B,), jnp.int32),
        scratch_shapes=[pltpu.VMEM((B,), jnp.float32)],
    )(content, bc3)


def _gather_rows_sc(table, idx):
    B = idx.shape[0]
    D = table.shape[1]
    n_workers = 16
    b_per_w = B // n_workers
    mesh = plsc.VectorSubcoreMesh(core_axis_name="c", subcore_axis_name="s")

    @functools.partial(
        pl.kernel,
        mesh=mesh,
        out_type=jax.ShapeDtypeStruct((B, D), jnp.float32),
        scratch_types=[
            pltpu.VMEM((b_per_w,), jnp.int32),
            pltpu.VMEM((b_per_w, D), jnp.float32),
            pltpu.SemaphoreType.DMA,
        ],
    )
    def k(table_hbm, idx_hbm, out_hbm, idx_v, rows_v, sem):
        wid = lax.axis_index("s") * 2 + lax.axis_index("c")

        @pl.when(wid < n_workers)
        def _():
            base = wid * b_per_w
            pltpu.sync_copy(idx_hbm.at[pl.ds(base, b_per_w)], idx_v)
            pltpu.async_copy(table_hbm.at[idx_v], rows_v, sem).wait()
            pltpu.sync_copy(rows_v, out_hbm.at[pl.ds(base, b_per_w)])

    return k(table, idx)


def kernel(content, bank_content, bank_style):
    B, D = content.shape
    M = bank_content.shape[0]
    G = M // D
    bc3 = bank_content.reshape(D, G, D)
    idx = _sim_argmax(content, bc3)
    return _gather_rows_sc(bank_style, idx)
